# baseline (speedup 1.0000x reference)
import dataclasses
import functools

import jax
import jax.numpy as jnp
from jax import lax
from jax.experimental import pallas as pl
from jax.experimental.pallas import tpu as pltpu
from jax.experimental.pallas import tpu_sc as plsc

BATCH = 20
SEQ = 8192
DIM = 64
N = BATCH * SEQ
VOCAB = 1000000

NUM_CORES = 2
NUM_SUBCORES = 16
NW = NUM_CORES * NUM_SUBCORES

_mesh = plsc.VectorSubcoreMesh(core_axis_name="c", subcore_axis_name="s")

_cp_a = pltpu.CompilerParams()
if "needs_layout_passes" in pltpu.CompilerParams.__dataclass_fields__:
    _cp_a = dataclasses.replace(_cp_a, needs_layout_passes=False)

FULL_TCOLS = VOCAB // 128
TAIL = VOCAB - FULL_TCOLS * 128
_BASE_TC = FULL_TCOLS // NW
_EXTRA_TC = FULL_TCOLS - _BASE_TC * NW
ABUF = 2


@functools.partial(
    pl.kernel,
    mesh=_mesh,
    compiler_params=_cp_a,
    out_type=jax.ShapeDtypeStruct((VOCAB * DIM,), jnp.float32),
    scratch_types=[
        *[pltpu.VMEM((64, 128), jnp.float32) for _ in range(ABUF)],
        *[pltpu.VMEM((64 * 128,), jnp.float32) for _ in range(ABUF)],
        *[pltpu.SemaphoreType.DMA for _ in range(ABUF)],
        *[pltpu.SemaphoreType.DMA for _ in range(ABUF)],
    ],
)
def _sc_transpose(tt_hbm, ttail_hbm, scr_hbm, *rest):
    bufs = rest[:ABUF]
    souts = rest[ABUF : 2 * ABUF]
    isems = rest[2 * ABUF : 3 * ABUF]
    osems = rest[3 * ABUF : 4 * ABUF]

    wid = lax.axis_index("s") * NUM_CORES + lax.axis_index("c")
    c0 = wid * _BASE_TC + jnp.minimum(wid, _EXTRA_TC)

    lane = lax.iota(jnp.int32, 16)
    dst = []
    for k in range(8):
        j = lane + 16 * k
        dst.append((j // 2) * 128 + 64 * (j % 2))

    def shuffle(buf, sout):
        def do_d(d, _):
            for k in range(8):
                v = buf[d, pl.ds(16 * k, 16)]
                plsc.store_scatter(sout, [dst[k] + d], v)
            return 0

        lax.fori_loop(0, 64, do_d, 0)

    def load(c, b):
        pltpu.async_copy(tt_hbm.at[:, pl.ds(c * 128, 128)], bufs[b], isems[b])

    def store(c, b):
        pltpu.async_copy(souts[b], scr_hbm.at[pl.ds(c * 8192, 8192)], osems[b])

    def wait_load(b):
        pltpu.make_async_copy(
            tt_hbm.at[:, pl.ds(0, 128)], bufs[b], isems[b]
        ).wait()

    def wait_store(b):
        pltpu.make_async_copy(
            souts[b], scr_hbm.at[pl.ds(0, 8192)], osems[b]
        ).wait()

    NG = _BASE_TC // ABUF
    for b in range(ABUF):
        load(c0 + b, b)
    for b in range(ABUF):
        wait_load(b)
        shuffle(bufs[b], souts[b])
        store(c0 + b, b)
        load(c0 + ABUF + b, b)

    def body(g, _):
        c = c0 + g * ABUF
        for b in range(ABUF):
            wait_load(b)
            wait_store(b)
            shuffle(bufs[b], souts[b])
            store(c + b, b)
            load(c + ABUF + b, b)
        return 0

    lax.fori_loop(1, NG - 1, body, 0)

    c_last = c0 + (NG - 1) * ABUF
    for b in range(ABUF):
        wait_load(b)
        wait_store(b)
        shuffle(bufs[b], souts[b])
        store(c_last + b, b)
    for b in range(ABUF):
        wait_store(b)

    @pl.when(wid < _EXTRA_TC)
    def _extra():
        c = c0 + _BASE_TC
        pltpu.sync_copy(tt_hbm.at[:, pl.ds(c * 128, 128)], bufs[0])
        shuffle(bufs[0], souts[0])
        pltpu.sync_copy(souts[0], scr_hbm.at[pl.ds(c * 8192, 8192)])

    @pl.when(wid == NW - 1)
    def _tail():
        pltpu.sync_copy(ttail_hbm, bufs[1])
        shuffle(bufs[1], souts[1])
        pltpu.sync_copy(souts[1], scr_hbm.at[pl.ds((VOCAB - 128) * DIM, 8192)])


PER_W = N // NW
CHUNK = 512
NCHUNK = PER_W // CHUNK
NBUF = 2


@functools.partial(
    pl.kernel,
    mesh=_mesh,
    compiler_params=pltpu.CompilerParams(use_tc_tiling_on_sc=False),
    out_type=jax.ShapeDtypeStruct((N, DIM), jnp.float32),
    scratch_types=[
        pltpu.VMEM((PER_W,), jnp.int32),
        *[pltpu.VMEM((CHUNK, DIM), jnp.float32) for _ in range(NBUF)],
        *[pltpu.SemaphoreType.DMA for _ in range(NBUF)],
        *[pltpu.SemaphoreType.DMA for _ in range(NBUF)],
    ],
)
def _sc_gather(idx_hbm, table_hbm, out_hbm, idx_v, *rest):
    bufs = rest[:NBUF]
    gsems = rest[NBUF : 2 * NBUF]
    ssems = rest[2 * NBUF : 3 * NBUF]

    wid = lax.axis_index("s") * NUM_CORES + lax.axis_index("c")
    base = wid * PER_W

    pltpu.sync_copy(idx_hbm.at[pl.ds(base, PER_W)], idx_v)

    gathers = [None] * NBUF
    scatters = [None] * NBUF
    for i in range(min(NBUF, NCHUNK)):
        gathers[i] = pltpu.async_copy(
            table_hbm.at[idx_v.at[pl.ds(i * CHUNK, CHUNK)]], bufs[i], gsems[i]
        )
    for i in range(NCHUNK):
        b = i % NBUF
        gathers[b].wait()
        scatters[b] = pltpu.async_copy(
            bufs[b], out_hbm.at[pl.ds(base + i * CHUNK, CHUNK)], ssems[b]
        )
        j = i + NBUF
        if j < NCHUNK:
            scatters[b].wait()
            gathers[b] = pltpu.async_copy(
                table_hbm.at[idx_v.at[pl.ds(j * CHUNK, CHUNK)]], bufs[b], gsems[b]
            )
    for i in range(max(0, NCHUNK - NBUF), NCHUNK):
        scatters[i % NBUF].wait()


def kernel(inputs, table):
    tt = table.T
    ttail = lax.slice(tt, (0, VOCAB - 128), (DIM, VOCAB))
    scr = _sc_transpose(tt, ttail)
    tbl = scr.reshape(VOCAB, DIM)
    flat_idx = inputs.reshape(-1)
    rows = _sc_gather(flat_idx, tbl)
    return rows.reshape(BATCH, -1)


# speedup vs baseline: 4.5421x; 4.5421x over previous
import dataclasses
import functools

import jax
import jax.numpy as jnp
from jax import lax
from jax.experimental import pallas as pl
from jax.experimental.pallas import tpu as pltpu
from jax.experimental.pallas import tpu_sc as plsc

BATCH = 20
SEQ = 8192
DIM = 64
N = BATCH * SEQ
VOCAB = 1000000

NUM_CORES = 2
NUM_SUBCORES = 16
NW = NUM_CORES * NUM_SUBCORES

_mesh = plsc.VectorSubcoreMesh(core_axis_name="c", subcore_axis_name="s")

_cp_a = pltpu.CompilerParams()
if "needs_layout_passes" in pltpu.CompilerParams.__dataclass_fields__:
    _cp_a = dataclasses.replace(_cp_a, needs_layout_passes=False)

FULL_TCOLS = VOCAB // 128
TAIL = VOCAB - FULL_TCOLS * 128
_BASE_TC = FULL_TCOLS // NW
_EXTRA_TC = FULL_TCOLS - _BASE_TC * NW
ABUF = 2


@functools.partial(
    pl.kernel,
    mesh=_mesh,
    compiler_params=_cp_a,
    out_type=jax.ShapeDtypeStruct((VOCAB * DIM,), jnp.float32),
    scratch_types=[
        *[pltpu.VMEM((64, 128), jnp.float32) for _ in range(ABUF)],
        *[pltpu.VMEM((64 * 128,), jnp.float32) for _ in range(ABUF)],
        *[pltpu.SemaphoreType.DMA for _ in range(ABUF)],
        *[pltpu.SemaphoreType.DMA for _ in range(ABUF)],
    ],
)
def _sc_transpose(tt_hbm, ttail_hbm, scr_hbm, *rest):
    bufs = rest[:ABUF]
    souts = rest[ABUF : 2 * ABUF]
    isems = rest[2 * ABUF : 3 * ABUF]
    osems = rest[3 * ABUF : 4 * ABUF]

    wid = lax.axis_index("s") * NUM_CORES + lax.axis_index("c")
    c0 = wid * _BASE_TC + jnp.minimum(wid, _EXTRA_TC)

    lane = lax.iota(jnp.int32, 16)
    dst = []
    for k in range(8):
        j = lane + 16 * k
        dst.append((j // 2) * 128 + 64 * (j % 2))

    def shuffle(buf, sout):
        @functools.partial(plsc.parallel_loop, 0, 64, unroll=4)
        def do_d(d):
            vs = [buf[d, pl.ds(16 * k, 16)] for k in range(8)]
            for k in range(8):
                plsc.store_scatter(sout, [dst[k] + d], vs[k])

    def load(c, b):
        pltpu.async_copy(tt_hbm.at[:, pl.ds(c * 128, 128)], bufs[b], isems[b])

    def store(c, b):
        pltpu.async_copy(souts[b], scr_hbm.at[pl.ds(c * 8192, 8192)], osems[b])

    def wait_load(b):
        pltpu.make_async_copy(
            tt_hbm.at[:, pl.ds(0, 128)], bufs[b], isems[b]
        ).wait()

    def wait_store(b):
        pltpu.make_async_copy(
            souts[b], scr_hbm.at[pl.ds(0, 8192)], osems[b]
        ).wait()

    NG = _BASE_TC // ABUF
    for b in range(ABUF):
        load(c0 + b, b)
    for b in range(ABUF):
        wait_load(b)
        shuffle(bufs[b], souts[b])
        store(c0 + b, b)
        load(c0 + ABUF + b, b)

    def body(g, _):
        c = c0 + g * ABUF
        for b in range(ABUF):
            wait_load(b)
            wait_store(b)
            shuffle(bufs[b], souts[b])
            store(c + b, b)
            load(c + ABUF + b, b)
        return 0

    lax.fori_loop(1, NG - 1, body, 0)

    c_last = c0 + (NG - 1) * ABUF
    for b in range(ABUF):
        wait_load(b)
        wait_store(b)
        shuffle(bufs[b], souts[b])
        store(c_last + b, b)
    for b in range(ABUF):
        wait_store(b)

    @pl.when(wid < _EXTRA_TC)
    def _extra():
        c = c0 + _BASE_TC
        pltpu.sync_copy(tt_hbm.at[:, pl.ds(c * 128, 128)], bufs[0])
        shuffle(bufs[0], souts[0])
        pltpu.sync_copy(souts[0], scr_hbm.at[pl.ds(c * 8192, 8192)])

    @pl.when(wid == NW - 1)
    def _tail():
        pltpu.sync_copy(ttail_hbm, bufs[1])
        shuffle(bufs[1], souts[1])
        pltpu.sync_copy(souts[1], scr_hbm.at[pl.ds((VOCAB - 128) * DIM, 8192)])


PER_W = N // NW
CHUNK = 512
NCHUNK = PER_W // CHUNK
NBUF = 2


@functools.partial(
    pl.kernel,
    mesh=_mesh,
    compiler_params=pltpu.CompilerParams(use_tc_tiling_on_sc=False),
    out_type=jax.ShapeDtypeStruct((N, DIM), jnp.float32),
    scratch_types=[
        pltpu.VMEM((PER_W,), jnp.int32),
        *[pltpu.VMEM((CHUNK, DIM), jnp.float32) for _ in range(NBUF)],
        *[pltpu.SemaphoreType.DMA for _ in range(NBUF)],
        *[pltpu.SemaphoreType.DMA for _ in range(NBUF)],
    ],
)
def _sc_gather(idx_hbm, table_hbm, out_hbm, idx_v, *rest):
    bufs = rest[:NBUF]
    gsems = rest[NBUF : 2 * NBUF]
    ssems = rest[2 * NBUF : 3 * NBUF]

    wid = lax.axis_index("s") * NUM_CORES + lax.axis_index("c")
    base = wid * PER_W

    pltpu.sync_copy(idx_hbm.at[pl.ds(base, PER_W)], idx_v)

    gathers = [None] * NBUF
    scatters = [None] * NBUF
    for i in range(min(NBUF, NCHUNK)):
        gathers[i] = pltpu.async_copy(
            table_hbm.at[idx_v.at[pl.ds(i * CHUNK, CHUNK)]], bufs[i], gsems[i]
        )
    for i in range(NCHUNK):
        b = i % NBUF
        gathers[b].wait()
        scatters[b] = pltpu.async_copy(
            bufs[b], out_hbm.at[pl.ds(base + i * CHUNK, CHUNK)], ssems[b]
        )
        j = i + NBUF
        if j < NCHUNK:
            scatters[b].wait()
            gathers[b] = pltpu.async_copy(
                table_hbm.at[idx_v.at[pl.ds(j * CHUNK, CHUNK)]], bufs[b], gsems[b]
            )
    for i in range(max(0, NCHUNK - NBUF), NCHUNK):
        scatters[i % NBUF].wait()


def kernel(inputs, table):
    tt = table.T
    ttail = lax.slice(tt, (0, VOCAB - 128), (DIM, VOCAB))
    scr = _sc_transpose(tt, ttail)
    tbl = scr.reshape(VOCAB, DIM)
    flat_idx = inputs.reshape(-1)
    rows = _sc_gather(flat_idx, tbl)
    return rows.reshape(BATCH, -1)
